# baseline (speedup 1.0000x reference)
import functools

import jax
import jax.numpy as jnp
from jax import lax
from jax.experimental import pallas as pl
from jax.experimental.pallas import tpu as pltpu
from jax.experimental.pallas import tpu_sc as plsc

EMB_DIM = 64
GROUP = 128

_info = plsc.get_sparse_core_info()
NC, NS = _info.num_cores, _info.num_subcores
NW = NC * NS


def _make_gather(n_groups_total):
    g_per_w = n_groups_total // NW
    mesh = plsc.VectorSubcoreMesh(core_axis_name="c", subcore_axis_name="s")

    @functools.partial(
        pl.kernel,
        mesh=mesh,
        out_type=jax.ShapeDtypeStruct((n_groups_total, GROUP, EMB_DIM), jnp.float32),
        compiler_params=pltpu.CompilerParams(use_tc_tiling_on_sc=False),
        scratch_types=[
            pltpu.VMEM((g_per_w, GROUP), jnp.int32),
            pltpu.VMEM((GROUP, EMB_DIM), jnp.float32),
            pltpu.SemaphoreType.DMA,
        ],
    )
    def gather_kernel(table_hbm, idx_hbm, out_hbm, idx_v, buf, sem):
        wid = lax.axis_index("s") * NC + lax.axis_index("c")
        base = wid * g_per_w
        pltpu.sync_copy(idx_hbm.at[wid], idx_v)

        def body(j, carry):
            pltpu.async_copy(table_hbm.at[idx_v.at[j]], buf, sem).wait()
            pltpu.sync_copy(buf, out_hbm.at[base + j])
            return carry

        lax.fori_loop(0, g_per_w, body, 0)

    return gather_kernel


_gather = _make_gather(1600)


def kernel(x, table):
    idx = x.reshape(NW, 1600 // NW, GROUP).astype(jnp.int32)
    out = _gather(table, idx)
    return out.reshape(x.shape + (EMB_DIM,))


# speedup vs baseline: 1.0460x; 1.0460x over previous
import functools

import jax
import jax.numpy as jnp
from jax import lax
from jax.experimental import pallas as pl
from jax.experimental.pallas import tpu as pltpu
from jax.experimental.pallas import tpu_sc as plsc

EMB_DIM = 64
GROUP = 128
NBUF = 5

_info = plsc.get_sparse_core_info()
NC, NS = _info.num_cores, _info.num_subcores
NW = NC * NS


def _make_gather(n_groups_total):
    g_per_w = n_groups_total // NW
    n_outer = g_per_w // NBUF
    mesh = plsc.VectorSubcoreMesh(core_axis_name="c", subcore_axis_name="s")

    @functools.partial(
        pl.kernel,
        mesh=mesh,
        out_type=jax.ShapeDtypeStruct((n_groups_total, GROUP, EMB_DIM), jnp.float32),
        compiler_params=pltpu.CompilerParams(use_tc_tiling_on_sc=False),
        scratch_types=[
            pltpu.VMEM((g_per_w, GROUP), jnp.int32),
            pltpu.VMEM((NBUF, GROUP, EMB_DIM), jnp.float32),
        ]
        + [pltpu.SemaphoreType.DMA] * (2 * NBUF),
    )
    def gather_kernel(table_hbm, idx_hbm, out_hbm, idx_v, bufs, *sems):
        gsem = sems[:NBUF]
        wsem = sems[NBUF:]
        wid = lax.axis_index("s") * NC + lax.axis_index("c")
        base = wid * g_per_w
        pltpu.sync_copy(idx_hbm.at[wid], idx_v)

        def gather(j, b):
            return pltpu.make_async_copy(
                table_hbm.at[idx_v.at[j]], bufs.at[b], gsem[b])

        def write(j, b):
            return pltpu.make_async_copy(
                bufs.at[b], out_hbm.at[base + j], wsem[b])

        for b in range(NBUF):
            gather(b, b).start()

        def outer(g, carry):
            for b in range(NBUF):
                j = g * NBUF + b
                gather(j, b).wait()
                write(j, b).start()
                nxt = j + NBUF

                @pl.when(nxt < g_per_w)
                def _():
                    write(j, b).wait()
                    gather(nxt, b).start()

            return carry

        lax.fori_loop(0, n_outer, outer, 0)
        for b in range(NBUF):
            write(g_per_w - NBUF + b, b).wait()

    return gather_kernel


_gather = _make_gather(1600)


def kernel(x, table):
    idx = x.reshape(NW, 1600 // NW, GROUP).astype(jnp.int32)
    out = _gather(table, idx)
    return out.reshape(x.shape + (EMB_DIM,))
